# Initial kernel scaffold; baseline (speedup 1.0000x reference)
import jax
import jax.numpy as jnp
from jax.experimental import pallas as pl


def kernel(x, edge_index, W1, b1, W2, b2, W3, b3, W4, b4, W5, b5, W6, b6):
    raise NotImplementedError("write your pallas kernel here")


# speedup vs baseline: 8.3028x; 8.3028x over previous
import functools

import jax
import jax.numpy as jnp
from jax import lax
from jax.experimental import pallas as pl
from jax.experimental.pallas import tpu as pltpu
from jax.experimental.pallas import tpu_sc as plsc

N = 10000
E = 320000
NC = 2
NS = 16
NW = NC * NS
B = 128
NCHUNK = 79
EPT = B * NCHUNK
EPAD = EPT * NW
GROW = N
NPADR = 10240
ACC_ROWS = NPADR
RPT = NPADR // NS

_mesh = plsc.VectorSubcoreMesh(core_axis_name="c", subcore_axis_name="s")


def _deg_body(dstp, ones_h, zeros_h, degp, dst_v, ones_v, acc, sem):
    cid = lax.axis_index("c")
    sid = lax.axis_index("s")
    wid = sid * NC + cid
    pltpu.sync_copy(ones_h, ones_v)
    pltpu.sync_copy(zeros_h, acc.at[pl.ds(sid * RPT, RPT)])
    plsc.subcore_barrier()
    base0 = wid * EPT

    def body(i, c):
        pltpu.sync_copy(dstp.at[pl.ds(base0 + i * B, B)], dst_v)
        pltpu.sync_copy(ones_v, acc.at[dst_v], add=True)
        return c

    lax.fori_loop(0, NCHUNK, body, 0)
    plsc.subcore_barrier()
    pltpu.sync_copy(acc.at[pl.ds(sid * RPT, RPT)],
                    degp.at[cid, pl.ds(sid * RPT, RPT)])


_deg = pl.kernel(
    _deg_body,
    out_type=jax.ShapeDtypeStruct((NC, NPADR, 128), jnp.float32),
    mesh=_mesh,
    scratch_types=[
        pltpu.VMEM((B,), jnp.int32),
        pltpu.VMEM((B, 128), jnp.float32),
        pltpu.VMEM_SHARED((ACC_ROWS, 128), jnp.float32),
        pltpu.SemaphoreType.DMA,
    ],
)


def _agg_body(table, srcp, dstp, zeros_h, part, src_v, dst_v, rows_v, acc, sem):
    cid = lax.axis_index("c")
    sid = lax.axis_index("s")
    wid = sid * NC + cid
    pltpu.sync_copy(zeros_h, acc.at[pl.ds(sid * RPT, RPT)])
    plsc.subcore_barrier()
    base0 = wid * EPT

    def body(i, c):
        base = base0 + i * B
        pltpu.sync_copy(srcp.at[pl.ds(base, B)], src_v)
        pltpu.sync_copy(dstp.at[pl.ds(base, B)], dst_v)
        pltpu.async_copy(table.at[src_v], rows_v, sem).wait()
        pltpu.sync_copy(rows_v, acc.at[dst_v], add=True)
        return c

    lax.fori_loop(0, NCHUNK, body, 0)
    plsc.subcore_barrier()
    pltpu.sync_copy(acc.at[pl.ds(sid * RPT, RPT)],
                    part.at[cid, pl.ds(sid * RPT, RPT)])


_agg = pl.kernel(
    _agg_body,
    out_type=jax.ShapeDtypeStruct((NC, NPADR, 128), jnp.float32),
    mesh=_mesh,
    scratch_types=[
        pltpu.VMEM((B,), jnp.int32),
        pltpu.VMEM((B,), jnp.int32),
        pltpu.VMEM((B, 128), jnp.float32),
        pltpu.VMEM_SHARED((ACC_ROWS, 128), jnp.float32),
        pltpu.SemaphoreType.DMA,
    ],
)


R = 1000
G = N // R


def _s_of(degp):
    return lax.rsqrt(degp[0, :, 0:1] + degp[1, :, 0:1] + 1.0)


def _tc0_body(degp, x, t0):
    t0[...] = x[...] * _s_of(degp[...])


def _tc1_body(degp, p1, t0, w1, b1, w2, t2):
    s = _s_of(degp[...])
    p = p1[...]
    a1 = s * (p[0] + p[1] + t0[...])
    h1 = jnp.maximum(
        jnp.dot(a1, w1[...].T, preferred_element_type=jnp.float32) + b1[...], 0.0)
    g2 = jnp.dot(h1, w2[...].T, preferred_element_type=jnp.float32)
    t2[...] = g2 * s


def _tc2_body(degp, p2a, p2b, t2, b2, w3, t3):
    s = _s_of(degp[...])
    pa = p2a[...]
    pb = p2b[...]
    agg = jnp.concatenate([pa[0] + pa[1], pb[0] + pb[1]], axis=1)
    h2 = jnp.maximum(s * (agg + t2[...]) + b2[...], 0.0)
    g3 = jnp.dot(h2, w3[...].T, preferred_element_type=jnp.float32)
    t3[...] = g3 * s


def _tc3_body(degp, p3, t3, b3, t4):
    s = _s_of(degp[...])
    p = p3[...]
    h3 = jnp.maximum(s * (p[0] + p[1] + t3[...]) + b3[...], 0.0)
    t4[...] = h3 * s


def _tc4_body(degp, p4, t4, w4, b4, t5):
    s = _s_of(degp[...])
    p = p4[...]
    a4 = s * (p[0] + p[1] + t4[...])
    h4 = jnp.maximum(
        jnp.dot(a4, w4[...].T, preferred_element_type=jnp.float32) + b4[...], 0.0)
    t5[...] = h4 * s


def _tc5_body(degp, p5a, p5b, t5, w5, b5, w6, t6):
    s = _s_of(degp[...])
    pa = p5a[...]
    pb = p5b[...]
    agg = jnp.concatenate([pa[0] + pa[1], pb[0] + pb[1]], axis=1)
    a5 = s * (agg + t5[...])
    h5 = jnp.maximum(
        jnp.dot(a5, w5[...].T, preferred_element_type=jnp.float32) + b5[...], 0.0)
    g6 = jnp.dot(h5, w6[...].T, preferred_element_type=jnp.float32)
    t6[...] = g6 * s


def _tc6_body(degp, p6, t6, b6, out):
    s = _s_of(degp[...])
    p = p6[...]
    out[...] = s * (p[0] + p[1] + t6[...]) + b6[...]


def _dspec():
    return pl.BlockSpec((NC, R, 128), lambda i: (0, i, 0))


def _pspec():
    return pl.BlockSpec((NC, R, 128), lambda i: (0, i, 0))


def _nspec(c):
    return pl.BlockSpec((R, c), lambda i: (i, 0))


def _wspec(a, b):
    return pl.BlockSpec((a, b), lambda i: (0, 0))


def _mk(body, in_specs, cout):
    return pl.pallas_call(
        body, grid=(G,), in_specs=in_specs, out_specs=_nspec(cout),
        out_shape=jax.ShapeDtypeStruct((N, cout), jnp.float32))


_tc0 = _mk(_tc0_body, [_dspec(), _nspec(128)], 128)
_tc1 = _mk(_tc1_body,
           [_dspec(), _pspec(), _nspec(128), _wspec(512, 128), _wspec(1, 512),
            _wspec(256, 512)], 256)
_tc2 = _mk(_tc2_body,
           [_dspec(), _pspec(), _pspec(), _nspec(256), _wspec(1, 256),
            _wspec(128, 256)], 128)
_tc3 = _mk(_tc3_body, [_dspec(), _pspec(), _nspec(128), _wspec(1, 128)], 128)
_tc4 = _mk(_tc4_body,
           [_dspec(), _pspec(), _nspec(128), _wspec(256, 128), _wspec(1, 256)],
           256)
_tc5 = _mk(_tc5_body,
           [_dspec(), _pspec(), _pspec(), _nspec(256), _wspec(512, 256),
            _wspec(1, 512), _wspec(128, 512)], 128)
_tc6 = _mk(_tc6_body, [_dspec(), _pspec(), _nspec(128), _wspec(1, 128)], 128)


def kernel(x, edge_index, W1, b1, W2, b2, W3, b3, W4, b4, W5, b5, W6, b6):
    src = edge_index[0].astype(jnp.int32)
    dst = edge_index[1].astype(jnp.int32)
    npad = EPAD - E
    srcp = jnp.concatenate([src, jnp.zeros((npad,), jnp.int32)])
    dstp = jnp.concatenate([dst, jnp.full((npad,), GROW, jnp.int32)])
    ones128 = jnp.ones((B, 128), jnp.float32)
    z128 = jnp.zeros((RPT, 128), jnp.float32)

    degp = _deg(dstp, ones128, z128)

    t0 = _tc0(degp, x)
    p1 = _agg(t0, srcp, dstp, z128)
    t2 = _tc1(degp, p1, t0, W1, b1.reshape(1, -1), W2)
    p2a = _agg(t2[:, :128], srcp, dstp, z128)
    p2b = _agg(t2[:, 128:], srcp, dstp, z128)
    t3 = _tc2(degp, p2a, p2b, t2, b2.reshape(1, -1), W3)
    p3 = _agg(t3, srcp, dstp, z128)
    t4 = _tc3(degp, p3, t3, b3.reshape(1, -1))
    p4 = _agg(t4, srcp, dstp, z128)
    t5 = _tc4(degp, p4, t4, W4, b4.reshape(1, -1))
    p5a = _agg(t5[:, :128], srcp, dstp, z128)
    p5b = _agg(t5[:, 128:], srcp, dstp, z128)
    t6 = _tc5(degp, p5a, p5b, t5, W5, b5.reshape(1, -1), W6)
    p6 = _agg(t6, srcp, dstp, z128)
    return _tc6(degp, p6, t6, b6.reshape(1, -1))
